# Initial kernel scaffold; baseline (speedup 1.0000x reference)
import jax
import jax.numpy as jnp
from jax.experimental import pallas as pl


def kernel(t_cls_0, t_cls_1, t_cls_2, t_cls_3, t_cls_4, t_bbox_0, t_bbox_1, t_bbox_2, t_bbox_3, t_bbox_4, t_angle_0, t_angle_1, t_angle_2, t_angle_3, t_angle_4, t_ctr_0, t_ctr_1, t_ctr_2, t_ctr_3, t_ctr_4, s_cls_0, s_cls_1, s_cls_2, s_cls_3, s_cls_4, s_bbox_0, s_bbox_1, s_bbox_2, s_bbox_3, s_bbox_4, s_angle_0, s_angle_1, s_angle_2, s_angle_3, s_angle_4, s_ctr_0, s_ctr_1, s_ctr_2, s_ctr_3, s_ctr_4):
    raise NotImplementedError("write your pallas kernel here")


# speedup vs baseline: 2.2857x; 2.2857x over previous
import functools

import jax
import jax.numpy as jnp
from jax import lax
from jax.experimental import pallas as pl
from jax.experimental.pallas import tpu as pltpu

_B = 16
_C = 16
_HW = (4096, 1024, 256, 64, 16)
_NPB = sum(_HW)
_N = _B * _NPB
_K = int(_N * 0.01)
_OFF = (0, 4096, 5120, 5376, 5440)


def _sigmoid(x):
    return jax.nn.sigmoid(x)


def _smooth_l1(x, y):
    d = jnp.abs(x - y)
    return jnp.where(d < 1.0, 0.5 * d * d, d - 0.5)


def _body(*refs):
    t_cls = refs[0:5]
    t_bbox = refs[5:10]
    t_angle = refs[10:15]
    t_ctr = refs[15:20]
    s_cls = refs[20:25]
    s_bbox = refs[25:30]
    s_angle = refs[30:35]
    s_ctr = refs[35:40]
    out_cls, out_bbox, out_ctr = refs[40:43]

    vs = []
    keys = []
    idxs = []
    for l in range(5):
        m = jnp.max(t_cls[l][...], axis=1)
        v = _sigmoid(m)
        vs.append(v)
        keys.append(lax.bitcast_convert_type(v, jnp.int32))
        bi = lax.broadcasted_iota(jnp.int32, v.shape, 0)
        pi = lax.broadcasted_iota(jnp.int32, v.shape, 1)
        idxs.append(bi * _NPB + (_OFF[l] + pi))

    def cnt_gt(x):
        c = jnp.int32(0)
        for k in keys:
            c = c + jnp.sum((k > x).astype(jnp.int32))
        return c

    def bstep(_, lohi):
        lo, hi = lohi
        mid = lo + (hi - lo) // 2
        take_hi = cnt_gt(mid) < _K
        return (jnp.where(take_hi, lo, mid), jnp.where(take_hi, mid, hi))

    lo0 = jnp.int32(-1)
    hi0 = jnp.int32(0x3F800000)
    _, t_key = lax.fori_loop(0, 31, bstep, (lo0, hi0))

    n_gt = cnt_gt(t_key)
    r = _K - n_gt

    def cnt_eq_le(x):
        c = jnp.int32(0)
        for k, ix in zip(keys, idxs):
            c = c + jnp.sum(((k == t_key) & (ix <= x)).astype(jnp.int32))
        return c

    def istep(_, lohi):
        lo, hi = lohi
        mid = lo + (hi - lo) // 2
        enough = cnt_eq_le(mid) >= r
        return (jnp.where(enough, lo, mid), jnp.where(enough, mid, hi))

    _, i_star = lax.fori_loop(0, 17, istep, (jnp.int32(-1), jnp.int32(_N - 1)))

    masks = []
    for l in range(5):
        masks.append((keys[l] > t_key) | ((keys[l] == t_key) & (idxs[l] <= i_star)))

    fg_num = jnp.float32(0.0)
    for l in range(5):
        fg_num = fg_num + jnp.sum(jnp.where(masks[l], vs[l], 0.0))

    acc_cls = jnp.float32(0.0)
    acc_bbox = jnp.float32(0.0)
    acc_ctr = jnp.float32(0.0)
    for l in range(5):
        mask = masks[l]
        p = _sigmoid(s_cls[l][...])
        t = _sigmoid(t_cls[l][...])
        pc = jnp.clip(p, 1e-12, 1.0 - 1e-12)
        lg1 = jnp.log(pc)
        lg0 = jnp.log(1.0 - pc)
        neg = -lg0 * (p * p)
        pos = -(t * lg1 + (1.0 - t) * lg0) * ((t - p) * (t - p))
        acc_cls = acc_cls + jnp.sum(jnp.where(mask[:, None, :], pos, neg))
        sl1 = jnp.sum(_smooth_l1(s_bbox[l][...], t_bbox[l][...]), axis=1)
        sl1 = sl1 + _smooth_l1(s_angle[l][...], t_angle[l][...])
        w = _sigmoid(t_ctr[l][...])
        acc_bbox = acc_bbox + jnp.sum(jnp.where(mask, sl1 * w, 0.0))
        ps = jnp.clip(_sigmoid(s_ctr[l][...]), 1e-12, 1.0 - 1e-12)
        bce = -(w * jnp.log(ps) + (1.0 - w) * jnp.log(1.0 - ps))
        acc_ctr = acc_ctr + jnp.sum(jnp.where(mask, bce, 0.0))

    out_cls[0, 0] = acc_cls / fg_num
    out_bbox[0, 0] = acc_bbox / jnp.float32(_K * 5)
    out_ctr[0, 0] = acc_ctr / jnp.float32(_K)


@jax.jit
def kernel(
    t_cls_0, t_cls_1, t_cls_2, t_cls_3, t_cls_4,
    t_bbox_0, t_bbox_1, t_bbox_2, t_bbox_3, t_bbox_4,
    t_angle_0, t_angle_1, t_angle_2, t_angle_3, t_angle_4,
    t_ctr_0, t_ctr_1, t_ctr_2, t_ctr_3, t_ctr_4,
    s_cls_0, s_cls_1, s_cls_2, s_cls_3, s_cls_4,
    s_bbox_0, s_bbox_1, s_bbox_2, s_bbox_3, s_bbox_4,
    s_angle_0, s_angle_1, s_angle_2, s_angle_3, s_angle_4,
    s_ctr_0, s_ctr_1, s_ctr_2, s_ctr_3, s_ctr_4,
):
    t_cls = [t_cls_0, t_cls_1, t_cls_2, t_cls_3, t_cls_4]
    t_bbox = [t_bbox_0, t_bbox_1, t_bbox_2, t_bbox_3, t_bbox_4]
    t_angle = [t_angle_0, t_angle_1, t_angle_2, t_angle_3, t_angle_4]
    t_ctr = [t_ctr_0, t_ctr_1, t_ctr_2, t_ctr_3, t_ctr_4]
    s_cls = [s_cls_0, s_cls_1, s_cls_2, s_cls_3, s_cls_4]
    s_bbox = [s_bbox_0, s_bbox_1, s_bbox_2, s_bbox_3, s_bbox_4]
    s_angle = [s_angle_0, s_angle_1, s_angle_2, s_angle_3, s_angle_4]
    s_ctr = [s_ctr_0, s_ctr_1, s_ctr_2, s_ctr_3, s_ctr_4]

    args = []
    for l in range(5):
        args.append(t_cls[l].reshape(_B, _C, _HW[l]))
    for l in range(5):
        args.append(t_bbox[l].reshape(_B, 4, _HW[l]))
    for l in range(5):
        args.append(t_angle[l].reshape(_B, _HW[l]))
    for l in range(5):
        args.append(t_ctr[l].reshape(_B, _HW[l]))
    for l in range(5):
        args.append(s_cls[l].reshape(_B, _C, _HW[l]))
    for l in range(5):
        args.append(s_bbox[l].reshape(_B, 4, _HW[l]))
    for l in range(5):
        args.append(s_angle[l].reshape(_B, _HW[l]))
    for l in range(5):
        args.append(s_ctr[l].reshape(_B, _HW[l]))

    out_shape = [jax.ShapeDtypeStruct((1, 1), jnp.float32)] * 3
    out_specs = [pl.BlockSpec(memory_space=pltpu.SMEM)] * 3
    loss_cls, loss_bbox, loss_ctr = pl.pallas_call(
        _body,
        out_shape=out_shape,
        out_specs=out_specs,
    )(*args)
    return (loss_cls.reshape(()), loss_bbox.reshape(()), loss_ctr.reshape(()))


# speedup vs baseline: 2.5909x; 1.1335x over previous
import functools

import jax
import jax.numpy as jnp
from jax import lax
from jax.experimental import pallas as pl
from jax.experimental.pallas import tpu as pltpu

_B = 16
_C = 16
_HW = (4096, 1024, 256, 64, 16)
_NPB = sum(_HW)
_N = _B * _NPB
_K = int(_N * 0.01)


def _sigmoid(x):
    return jax.nn.sigmoid(x)


def _smooth_l1(x, y):
    d = jnp.abs(x - y)
    return jnp.where(d < 1.0, 0.5 * d * d, d - 0.5)


def _vbody(t_cls_ref, v_ref):
    v_ref[...] = _sigmoid(jnp.max(t_cls_ref[...], axis=1))[:, None, :]


def _mask_for(v_blk, b, t_key, i_star):
    key = lax.bitcast_convert_type(v_blk, jnp.int32)
    idx = b * _NPB + lax.broadcasted_iota(jnp.int32, v_blk.shape, 2)
    return (key > t_key) | ((key == t_key) & (idx <= i_star))


def _lbody(v_ref, t_cls_ref, s_cls_ref, t_bb5_ref, s_bb5_ref, t_ctr_ref,
           s_ctr_ref, out_cls, out_bbox, out_ctr, sdi, sdf):
    i = pl.program_id(0)

    @pl.when(i == 0)
    def _search():
        v = v_ref[...]
        keys = lax.bitcast_convert_type(v, jnp.int32)
        idxs = (lax.broadcasted_iota(jnp.int32, v.shape, 0) * _NPB
                + lax.broadcasted_iota(jnp.int32, v.shape, 2))

        def cnt_gt(x):
            return jnp.sum((keys > x).astype(jnp.int32))

        def bstep(_, lohi):
            lo, hi = lohi
            mid = lo + (hi - lo) // 2
            take_hi = cnt_gt(mid) < _K
            return (jnp.where(take_hi, lo, mid), jnp.where(take_hi, mid, hi))

        _, t_key = lax.fori_loop(0, 31, bstep,
                                 (jnp.int32(-1), jnp.int32(0x3F800000)))
        r = _K - cnt_gt(t_key)

        eq = keys == t_key

        def cnt_eq_le(x):
            return jnp.sum((eq & (idxs <= x)).astype(jnp.int32))

        def istep(_, lohi):
            lo, hi = lohi
            mid = lo + (hi - lo) // 2
            enough = cnt_eq_le(mid) >= r
            return (jnp.where(enough, lo, mid), jnp.where(enough, mid, hi))

        _, i_star = lax.fori_loop(0, 17, istep,
                                  (jnp.int32(-1), jnp.int32(_N - 1)))

        t_val = lax.bitcast_convert_type(t_key, jnp.float32)
        fg_num = jnp.sum(jnp.where(keys > t_key, v, 0.0)) \
            + t_val * r.astype(jnp.float32)
        sdi[0] = t_key
        sdi[1] = i_star
        sdf[0] = fg_num
        sdf[1] = jnp.float32(0.0)
        sdf[2] = jnp.float32(0.0)
        sdf[3] = jnp.float32(0.0)

    @pl.when(i > 0)
    def _losses():
        b = i - 1
        t_key = sdi[0]
        i_star = sdi[1]
        v_blk = v_ref[pl.ds(b, 1), :, :]
        mask = _mask_for(v_blk, b, t_key, i_star)

        p = _sigmoid(s_cls_ref[...])
        t = _sigmoid(t_cls_ref[...])
        pc = jnp.clip(p, 1e-12, 1.0 - 1e-12)
        lg1 = jnp.log(pc)
        lg0 = jnp.log(1.0 - pc)
        neg = -lg0 * (p * p)
        pos = -(t * lg1 + (1.0 - t) * lg0) * ((t - p) * (t - p))
        sdf[1] += jnp.sum(jnp.where(mask, pos, neg))

        mask2 = mask[:, 0, :]
        sl1 = jnp.sum(_smooth_l1(s_bb5_ref[...], t_bb5_ref[...]), axis=1)
        w = _sigmoid(t_ctr_ref[...])[:, 0, :]
        sdf[2] += jnp.sum(jnp.where(mask2, sl1 * w, 0.0))

        ps = jnp.clip(_sigmoid(s_ctr_ref[...])[:, 0, :], 1e-12, 1.0 - 1e-12)
        bce = -(w * jnp.log(ps) + (1.0 - w) * jnp.log(1.0 - ps))
        sdf[3] += jnp.sum(jnp.where(mask2, bce, 0.0))

    @pl.when(i == pl.num_programs(0) - 1)
    def _finish():
        out_cls[0, 0] = sdf[1] / sdf[0]
        out_bbox[0, 0] = sdf[2] / jnp.float32(_K * 5)
        out_ctr[0, 0] = sdf[3] / jnp.float32(_K)


def _pack(cls_l, bbox_l, angle_l, ctr_l):
    cls = jnp.concatenate(
        [x.reshape(_B, _C, -1) for x in cls_l], axis=2)
    bb5 = jnp.concatenate(
        [jnp.concatenate([x.reshape(_B, 4, -1), y.reshape(_B, 1, -1)],
                         axis=1) for x, y in zip(bbox_l, angle_l)], axis=2)
    ctr = jnp.concatenate([x.reshape(_B, 1, -1) for x in ctr_l], axis=2)
    return cls, bb5, ctr


@jax.jit
def kernel(
    t_cls_0, t_cls_1, t_cls_2, t_cls_3, t_cls_4,
    t_bbox_0, t_bbox_1, t_bbox_2, t_bbox_3, t_bbox_4,
    t_angle_0, t_angle_1, t_angle_2, t_angle_3, t_angle_4,
    t_ctr_0, t_ctr_1, t_ctr_2, t_ctr_3, t_ctr_4,
    s_cls_0, s_cls_1, s_cls_2, s_cls_3, s_cls_4,
    s_bbox_0, s_bbox_1, s_bbox_2, s_bbox_3, s_bbox_4,
    s_angle_0, s_angle_1, s_angle_2, s_angle_3, s_angle_4,
    s_ctr_0, s_ctr_1, s_ctr_2, s_ctr_3, s_ctr_4,
):
    t_cls, t_bb5, t_ctr = _pack(
        [t_cls_0, t_cls_1, t_cls_2, t_cls_3, t_cls_4],
        [t_bbox_0, t_bbox_1, t_bbox_2, t_bbox_3, t_bbox_4],
        [t_angle_0, t_angle_1, t_angle_2, t_angle_3, t_angle_4],
        [t_ctr_0, t_ctr_1, t_ctr_2, t_ctr_3, t_ctr_4])
    s_cls, s_bb5, s_ctr = _pack(
        [s_cls_0, s_cls_1, s_cls_2, s_cls_3, s_cls_4],
        [s_bbox_0, s_bbox_1, s_bbox_2, s_bbox_3, s_bbox_4],
        [s_angle_0, s_angle_1, s_angle_2, s_angle_3, s_angle_4],
        [s_ctr_0, s_ctr_1, s_ctr_2, s_ctr_3, s_ctr_4])

    v = pl.pallas_call(
        _vbody,
        grid=(_B,),
        in_specs=[pl.BlockSpec((1, _C, _NPB), lambda b: (b, 0, 0))],
        out_specs=pl.BlockSpec((1, 1, _NPB), lambda b: (b, 0, 0)),
        out_shape=jax.ShapeDtypeStruct((_B, 1, _NPB), jnp.float32),
    )(t_cls)

    def bmap(i):
        b = jnp.clip(i - 1, 0, _B - 1)
        return b

    loss_cls, loss_bbox, loss_ctr = pl.pallas_call(
        _lbody,
        grid=(_B + 1,),
        in_specs=[
            pl.BlockSpec((_B, 1, _NPB), lambda i: (0, 0, 0)),
            pl.BlockSpec((1, _C, _NPB), lambda i: (bmap(i), 0, 0)),
            pl.BlockSpec((1, _C, _NPB), lambda i: (bmap(i), 0, 0)),
            pl.BlockSpec((1, 5, _NPB), lambda i: (bmap(i), 0, 0)),
            pl.BlockSpec((1, 5, _NPB), lambda i: (bmap(i), 0, 0)),
            pl.BlockSpec((1, 1, _NPB), lambda i: (bmap(i), 0, 0)),
            pl.BlockSpec((1, 1, _NPB), lambda i: (bmap(i), 0, 0)),
        ],
        out_specs=[pl.BlockSpec(memory_space=pltpu.SMEM)] * 3,
        out_shape=[jax.ShapeDtypeStruct((1, 1), jnp.float32)] * 3,
        scratch_shapes=[pltpu.SMEM((4,), jnp.int32),
                        pltpu.SMEM((4,), jnp.float32)],
    )(v, t_cls, s_cls, t_bb5, s_bb5, t_ctr, s_ctr)
    return (loss_cls.reshape(()), loss_bbox.reshape(()), loss_ctr.reshape(()))


# speedup vs baseline: 3.6453x; 1.4070x over previous
import jax
import jax.numpy as jnp
from jax import lax
from jax.experimental import pallas as pl
from jax.experimental.pallas import tpu as pltpu

_B = 16
_C = 16
_SZ = ((64, 64), (32, 32), (16, 16), (8, 8), (4, 4))
_HW = tuple(h * w for h, w in _SZ)
_NPB = sum(_HW)
_N = _B * _NPB
_K = int(_N * 0.01)
_OFF = (0, 4096, 5120, 5376, 5440)


def _smooth_l1(x, y):
    d = jnp.abs(x - y)
    return jnp.where(d < 1.0, 0.5 * d * d, d - 0.5)


def _vbody(*refs):
    t_cls = refs[0:5]
    vout = refs[5:10]
    for l in range(5):
        vout[l][...] = jax.nn.sigmoid(jnp.max(t_cls[l][...], axis=1))


def _idx3(shape, l, b):
    w = _SZ[l][1]
    yi = lax.broadcasted_iota(jnp.int32, shape, 1)
    xi = lax.broadcasted_iota(jnp.int32, shape, 2)
    return b * _NPB + _OFF[l] + yi * w + xi


def _lbody(*refs):
    v = refs[0:5]
    t_cls = refs[5:10]
    s_cls = refs[10:15]
    t_bbox = refs[15:20]
    t_angle = refs[20:25]
    t_ctr = refs[25:30]
    s_bbox = refs[30:35]
    s_angle = refs[35:40]
    s_ctr = refs[40:45]
    out_cls, out_bbox, out_ctr = refs[45:48]
    sdi, sdf = refs[48:50]

    i = pl.program_id(0)

    @pl.when(i == 0)
    def _search():
        keys = [lax.bitcast_convert_type(v[l][...], jnp.int32)
                for l in range(5)]

        def cnt_gt(x):
            c = jnp.int32(0)
            for k in keys:
                c = c + jnp.sum((k > x).astype(jnp.int32))
            return c

        def bstep(_, lohi):
            lo, hi = lohi
            mid = lo + (hi - lo) // 2
            take_hi = cnt_gt(mid) < _K
            return (jnp.where(take_hi, lo, mid), jnp.where(take_hi, mid, hi))

        _, t_key = lax.fori_loop(0, 31, bstep,
                                 (jnp.int32(-1), jnp.int32(0x3F800000)))
        r = _K - cnt_gt(t_key)

        def cnt_eq_le(x):
            c = jnp.int32(0)
            for l in range(5):
                idx = (lax.broadcasted_iota(jnp.int32, keys[l].shape, 0)
                       * _NPB + _OFF[l]
                       + lax.broadcasted_iota(jnp.int32, keys[l].shape, 1)
                       * _SZ[l][1]
                       + lax.broadcasted_iota(jnp.int32, keys[l].shape, 2))
                c = c + jnp.sum(((keys[l] == t_key) & (idx <= x))
                                .astype(jnp.int32))
            return c

        def istep(_, lohi):
            lo, hi = lohi
            mid = lo + (hi - lo) // 2
            enough = cnt_eq_le(mid) >= r
            return (jnp.where(enough, lo, mid), jnp.where(enough, mid, hi))

        _, i_star = lax.fori_loop(0, 17, istep,
                                  (jnp.int32(-1), jnp.int32(_N - 1)))

        t_val = lax.bitcast_convert_type(t_key, jnp.float32)
        fg = jnp.float32(0.0)
        for l in range(5):
            fg = fg + jnp.sum(jnp.where(keys[l] > t_key, v[l][...], 0.0))
        sdi[0] = t_key
        sdi[1] = i_star
        sdf[0] = fg + t_val * r.astype(jnp.float32)
        sdf[1] = jnp.float32(0.0)
        sdf[2] = jnp.float32(0.0)
        sdf[3] = jnp.float32(0.0)

    @pl.when(i > 0)
    def _losses():
        b = i - 1
        t_key = sdi[0]
        i_star = sdi[1]
        acc_cls = jnp.float32(0.0)
        acc_bbox = jnp.float32(0.0)
        acc_ctr = jnp.float32(0.0)
        for l in range(5):
            v_blk = v[l][pl.ds(b, 1), :, :]
            key = lax.bitcast_convert_type(v_blk, jnp.int32)
            idx = _idx3(v_blk.shape, l, b)
            mask = (key > t_key) | ((key == t_key) & (idx <= i_star))

            x = s_cls[l][...]
            e = jnp.exp(-jnp.abs(x))
            a = 1.0 + e
            p = jnp.where(x >= 0.0, 1.0, e) / a
            sp = jnp.maximum(x, 0.0) + jnp.log(a)
            t = jax.nn.sigmoid(t_cls[l][...])
            d = t - p
            pos = (sp - t * x) * (d * d)
            neg = sp * (p * p)
            acc_cls = acc_cls + jnp.sum(
                jnp.where(mask[:, None, :, :], pos, neg))

            sl1 = jnp.sum(_smooth_l1(s_bbox[l][...], t_bbox[l][...]), axis=1)
            sl1 = sl1 + _smooth_l1(s_angle[l][...][:, 0], t_angle[l][...][:, 0])
            w = jax.nn.sigmoid(t_ctr[l][...][:, 0])
            acc_bbox = acc_bbox + jnp.sum(jnp.where(mask, sl1 * w, 0.0))

            xs = s_ctr[l][...][:, 0]
            es = jnp.exp(-jnp.abs(xs))
            sps = jnp.maximum(xs, 0.0) + jnp.log(1.0 + es)
            acc_ctr = acc_ctr + jnp.sum(jnp.where(mask, sps - w * xs, 0.0))

        sdf[1] += acc_cls
        sdf[2] += acc_bbox
        sdf[3] += acc_ctr

    @pl.when(i == pl.num_programs(0) - 1)
    def _finish():
        out_cls[0, 0] = sdf[1] / sdf[0]
        out_bbox[0, 0] = sdf[2] / jnp.float32(_K * 5)
        out_ctr[0, 0] = sdf[3] / jnp.float32(_K)


@jax.jit
def kernel(
    t_cls_0, t_cls_1, t_cls_2, t_cls_3, t_cls_4,
    t_bbox_0, t_bbox_1, t_bbox_2, t_bbox_3, t_bbox_4,
    t_angle_0, t_angle_1, t_angle_2, t_angle_3, t_angle_4,
    t_ctr_0, t_ctr_1, t_ctr_2, t_ctr_3, t_ctr_4,
    s_cls_0, s_cls_1, s_cls_2, s_cls_3, s_cls_4,
    s_bbox_0, s_bbox_1, s_bbox_2, s_bbox_3, s_bbox_4,
    s_angle_0, s_angle_1, s_angle_2, s_angle_3, s_angle_4,
    s_ctr_0, s_ctr_1, s_ctr_2, s_ctr_3, s_ctr_4,
):
    t_cls = [t_cls_0, t_cls_1, t_cls_2, t_cls_3, t_cls_4]
    t_bbox = [t_bbox_0, t_bbox_1, t_bbox_2, t_bbox_3, t_bbox_4]
    t_angle = [t_angle_0, t_angle_1, t_angle_2, t_angle_3, t_angle_4]
    t_ctr = [t_ctr_0, t_ctr_1, t_ctr_2, t_ctr_3, t_ctr_4]
    s_cls = [s_cls_0, s_cls_1, s_cls_2, s_cls_3, s_cls_4]
    s_bbox = [s_bbox_0, s_bbox_1, s_bbox_2, s_bbox_3, s_bbox_4]
    s_angle = [s_angle_0, s_angle_1, s_angle_2, s_angle_3, s_angle_4]
    s_ctr = [s_ctr_0, s_ctr_1, s_ctr_2, s_ctr_3, s_ctr_4]

    v_specs_in = [pl.BlockSpec((1, _C, h, w), lambda b: (b, 0, 0, 0))
                  for h, w in _SZ]
    v_specs_out = [pl.BlockSpec((1, h, w), lambda b: (b, 0, 0))
                   for h, w in _SZ]
    v = pl.pallas_call(
        _vbody,
        grid=(_B,),
        in_specs=v_specs_in,
        out_specs=v_specs_out,
        out_shape=[jax.ShapeDtypeStruct((_B, h, w), jnp.float32)
                   for h, w in _SZ],
    )(*t_cls)

    def bmap4(i):
        return (jnp.clip(i - 1, 0, _B - 1), 0, 0, 0)

    full3 = [pl.BlockSpec((_B, h, w), lambda i: (0, 0, 0)) for h, w in _SZ]
    blk_cls = [pl.BlockSpec((1, _C, h, w), bmap4) for h, w in _SZ]
    blk_bb = [pl.BlockSpec((1, 4, h, w), bmap4) for h, w in _SZ]
    blk_1 = [pl.BlockSpec((1, 1, h, w), bmap4) for h, w in _SZ]

    loss_cls, loss_bbox, loss_ctr = pl.pallas_call(
        _lbody,
        grid=(_B + 1,),
        in_specs=(full3 + blk_cls + blk_cls + blk_bb + blk_1 + blk_1
                  + blk_bb + blk_1 + blk_1),
        out_specs=[pl.BlockSpec(memory_space=pltpu.SMEM)] * 3,
        out_shape=[jax.ShapeDtypeStruct((1, 1), jnp.float32)] * 3,
        scratch_shapes=[pltpu.SMEM((4,), jnp.int32),
                        pltpu.SMEM((4,), jnp.float32)],
    )(*(list(v) + t_cls + s_cls + t_bbox + t_angle + t_ctr
        + s_bbox + s_angle + s_ctr))
    return (loss_cls.reshape(()), loss_bbox.reshape(()), loss_ctr.reshape(()))
